# baseline (speedup 1.0000x reference)
import functools

import jax
import jax.numpy as jnp
from jax import lax
from jax.experimental import pallas as pl
from jax.experimental.pallas import tpu as pltpu
from jax.experimental.pallas import tpu_sc as plsc

_NC, _NS = 2, 16
_NW = _NC * _NS
_L = 16

_SC_PARAMS = pltpu.CompilerParams(
    needs_layout_passes=False, use_tc_tiling_on_sc=False)


def _matmul_ht(x, w):
    n, d = x.shape
    h = w.shape[1]
    bm = 4096
    def body(x_ref, w_ref, o_ref):
        r = jnp.dot(x_ref[...], w_ref[...],
                    preferred_element_type=jnp.float32)
        r3 = r.reshape(bm // 8, 8, h)
        o_ref[...] = jnp.concatenate([r3[:, k, :] for k in range(8)],
                                     axis=1)
    return pl.pallas_call(
        body,
        grid=((n + bm - 1) // bm,),
        in_specs=[pl.BlockSpec((bm, d), lambda i: (i, 0)),
                  pl.BlockSpec((d, h), lambda i: (0, 0))],
        out_specs=pl.BlockSpec((bm // 8, 8 * h), lambda i: (i, 0)),
        out_shape=jax.ShapeDtypeStruct((n // 8, 8 * h), jnp.float32),
    )(x, w)


def _make_prep(n_src, n_rows1, n_rows2, n1_dst, n2_dst):
    rows1_pt = n_rows1 // _NW
    rows2_pt = n_rows2 // _NW
    d1_pt = n1_dst // _NS
    d2_pt = n2_dst // _NS
    mesh = plsc.VectorSubcoreMesh(
        core_axis_name="c", subcore_axis_name="s",
        num_cores=_NC, num_subcores=_NS)
    scratch = [
        pltpu.VMEM((rows1_pt, 128), jnp.int32),
        pltpu.VMEM((rows1_pt, 128), jnp.int32),
        pltpu.VMEM((rows2_pt, 128), jnp.int32),
        pltpu.VMEM((128, _L), jnp.float32),
        pltpu.VMEM((d1_pt, _L), jnp.float32),
        pltpu.VMEM((n_src,), jnp.int32),
        pltpu.VMEM_SHARED((n1_dst, _L), jnp.float32),
        pltpu.VMEM_SHARED((n2_dst, _L), jnp.float32),
        pltpu.SemaphoreType.DMA,
    ]
    out_type = (jax.ShapeDtypeStruct((n_rows1, 128), jnp.int32),
                jax.ShapeDtypeStruct((_NC * n1_dst, _L), jnp.float32),
                jax.ShapeDtypeStruct((_NC * n2_dst, _L), jnp.float32))

    @functools.partial(pl.kernel, mesh=mesh, out_type=out_type,
                       scratch_types=scratch, compiler_params=_SC_PARAMS)
    def prep(srcr, nidr, dst1r, dst2r, zeros_h, ones_h,
             idx_out, c1_out, c2_out,
             src_v, dst1_v, dst2_v, ones_v, stage_v, nid_v,
             cnt1, cnt2, sem):
        cx = lax.axis_index("c")
        sx = lax.axis_index("s")
        w = cx * _NS + sx
        base1 = w * rows1_pt
        base2 = w * rows2_pt
        pltpu.sync_copy(ones_h, ones_v)
        pltpu.sync_copy(srcr.at[pl.ds(base1, rows1_pt)], src_v)
        pltpu.sync_copy(dst1r.at[pl.ds(base1, rows1_pt)], dst1_v)
        pltpu.sync_copy(dst2r.at[pl.ds(base2, rows2_pt)], dst2_v)
        pltpu.sync_copy(nidr, nid_v)
        z1 = sx * d1_pt
        z2 = sx * d2_pt
        pltpu.sync_copy(zeros_h.at[pl.ds(z1, d1_pt)],
                        cnt1.at[pl.ds(z1, d1_pt)])
        pltpu.sync_copy(zeros_h.at[pl.ds(z2, d2_pt)],
                        cnt2.at[pl.ds(z2, d2_pt)])
        plsc.subcore_barrier()

        for r in range(rows1_pt):
            pltpu.async_copy(ones_v, cnt1.at[dst1_v.at[r]], sem, add=True)
        for r in range(rows2_pt):
            pltpu.async_copy(ones_v, cnt2.at[dst2_v.at[r]], sem, add=True)

        def comp(r, carry):
            for t in range(8):
                vec = src_v[r, pl.ds(t * 16, 16)]
                src_v[r, pl.ds(t * 16, 16)] = plsc.load_gather(nid_v, [vec])
            return carry
        lax.fori_loop(0, rows1_pt, comp, 0)
        pltpu.sync_copy(src_v, idx_out.at[pl.ds(base1, rows1_pt)])

        for r in range(rows1_pt):
            pltpu.make_async_copy(ones_v, cnt1.at[dst1_v.at[r]],
                                  sem).wait()
        for r in range(rows2_pt):
            pltpu.make_async_copy(ones_v, cnt2.at[dst2_v.at[r]],
                                  sem).wait()
        plsc.subcore_barrier()

        o1 = cx * n1_dst + z1
        o2 = cx * n2_dst + z2
        pltpu.sync_copy(cnt1.at[pl.ds(z1, d1_pt)],
                        stage_v.at[pl.ds(0, d1_pt)])
        pltpu.sync_copy(stage_v.at[pl.ds(0, d1_pt)],
                        c1_out.at[pl.ds(o1, d1_pt)])
        pltpu.sync_copy(cnt2.at[pl.ds(z2, d2_pt)],
                        stage_v.at[pl.ds(0, d2_pt)])
        pltpu.sync_copy(stage_v.at[pl.ds(0, d2_pt)],
                        c2_out.at[pl.ds(o2, d2_pt)])

    return prep


def _make_agg1(n_dst, rows_pt, kb):
    n_chunks = rows_pt // kb
    dst_pt = n_dst // _NS
    mesh = plsc.VectorSubcoreMesh(
        core_axis_name="c", subcore_axis_name="s",
        num_cores=_NC, num_subcores=_NS)

    scratch = [
        pltpu.VMEM((rows_pt, 128), jnp.int32),
        pltpu.VMEM((rows_pt, 128), jnp.int32),
        pltpu.VMEM((kb * 128, _L), jnp.float32),
        pltpu.VMEM((kb * 128, _L), jnp.float32),
        pltpu.VMEM_SHARED((n_dst, _L), jnp.float32),
        pltpu.SemaphoreType.DMA,
        pltpu.SemaphoreType.DMA,
    ]
    out_type = jax.ShapeDtypeStruct((_NC * n_dst, _L), jnp.float32)

    @functools.partial(pl.kernel, mesh=mesh, out_type=out_type,
                       scratch_types=scratch, compiler_params=_SC_PARAMS)
    def agg(table, idxr, dstr, zeros_h, s_out,
            src_v, dst_v, rows_a, rows_b, acc, gsem, ssem):
        cx = lax.axis_index("c")
        sx = lax.axis_index("s")
        w = cx * _NS + sx
        base = w * rows_pt

        pltpu.sync_copy(idxr.at[pl.ds(base, rows_pt)], src_v)
        pltpu.sync_copy(dstr.at[pl.ds(base, rows_pt)], dst_v)
        z0 = sx * dst_pt
        pltpu.sync_copy(zeros_h.at[pl.ds(z0, dst_pt)],
                        acc.at[pl.ds(z0, dst_pt)])
        plsc.subcore_barrier()

        def fire_g(c, buf):
            for r in range(kb):
                pltpu.async_copy(table.at[src_v.at[c * kb + r]],
                                 buf.at[pl.ds(r * 128, 128)], gsem)

        def drain_g(c, buf):
            for r in range(kb):
                pltpu.make_async_copy(
                    table.at[src_v.at[c * kb + r]],
                    buf.at[pl.ds(r * 128, 128)], gsem).wait()

        def fire_s(c, buf):
            for r in range(kb):
                pltpu.async_copy(buf.at[pl.ds(r * 128, 128)],
                                 acc.at[dst_v.at[c * kb + r]], ssem,
                                 add=True)

        def drain_s(c, buf):
            for r in range(kb):
                pltpu.make_async_copy(
                    buf.at[pl.ds(r * 128, 128)],
                    acc.at[dst_v.at[c * kb + r]], ssem).wait()

        fire_g(0, rows_a)
        fire_g(1, rows_b)
        drain_g(0, rows_a)
        fire_s(0, rows_a)

        def pair(i, carry):
            c = 1 + 2 * i
            drain_s(c - 1, rows_a)
            fire_g(c + 1, rows_a)
            drain_g(c, rows_b)
            fire_s(c, rows_b)
            drain_s(c, rows_b)
            fire_g(c + 2, rows_b)
            drain_g(c + 1, rows_a)
            fire_s(c + 1, rows_a)
            return carry
        lax.fori_loop(0, (n_chunks - 2) // 2, pair, 0)

        last = n_chunks - 1
        drain_s(last - 1, rows_a)
        drain_g(last, rows_b)
        fire_s(last, rows_b)
        drain_s(last, rows_b)
        plsc.subcore_barrier()

        o0 = cx * n_dst + sx * dst_pt
        pltpu.sync_copy(acc.at[pl.ds(z0, dst_pt)],
                        rows_a.at[pl.ds(0, dst_pt)])
        pltpu.sync_copy(rows_a.at[pl.ds(0, dst_pt)],
                        s_out.at[pl.ds(o0, dst_pt)])

    return agg


def _make_agg2(n_dst, rows_pt):
    dst_pt = n_dst // _NS
    rows_cap = max(rows_pt * 128, dst_pt)
    mesh = plsc.VectorSubcoreMesh(
        core_axis_name="c", subcore_axis_name="s",
        num_cores=_NC, num_subcores=_NS)

    scratch = [
        pltpu.VMEM((rows_pt, 128), jnp.int32),
        pltpu.VMEM((rows_pt, 128), jnp.int32),
        pltpu.VMEM((rows_cap, _L), jnp.float32),
        pltpu.VMEM_SHARED((n_dst, _L), jnp.float32),
        pltpu.SemaphoreType.DMA,
        pltpu.SemaphoreType.DMA,
    ]
    out_type = jax.ShapeDtypeStruct((_NC * n_dst, _L), jnp.float32)

    @functools.partial(pl.kernel, mesh=mesh, out_type=out_type,
                       scratch_types=scratch, compiler_params=_SC_PARAMS)
    def agg(table, srcr, dstr, zeros_h, s_out,
            src_v, dst_v, rows_v, acc, gsem, ssem):
        cx = lax.axis_index("c")
        sx = lax.axis_index("s")
        w = cx * _NS + sx
        base = w * rows_pt

        pltpu.sync_copy(srcr.at[pl.ds(base, rows_pt)], src_v)
        pltpu.sync_copy(dstr.at[pl.ds(base, rows_pt)], dst_v)
        z0 = sx * dst_pt
        pltpu.sync_copy(zeros_h.at[pl.ds(z0, dst_pt)],
                        acc.at[pl.ds(z0, dst_pt)])
        plsc.subcore_barrier()

        for r in range(rows_pt):
            pltpu.async_copy(table.at[src_v.at[r]],
                             rows_v.at[pl.ds(r * 128, 128)], gsem)
        for r in range(rows_pt):
            pltpu.make_async_copy(table.at[src_v.at[r]],
                                  rows_v.at[pl.ds(r * 128, 128)],
                                  gsem).wait()
        for r in range(rows_pt):
            pltpu.async_copy(rows_v.at[pl.ds(r * 128, 128)],
                             acc.at[dst_v.at[r]], ssem, add=True)
        for r in range(rows_pt):
            pltpu.make_async_copy(rows_v.at[pl.ds(r * 128, 128)],
                                  acc.at[dst_v.at[r]], ssem).wait()
        plsc.subcore_barrier()

        o0 = cx * n_dst + sx * dst_pt
        pltpu.sync_copy(acc.at[pl.ds(z0, dst_pt)],
                        rows_v.at[pl.ds(0, dst_pt)])
        pltpu.sync_copy(rows_v.at[pl.ds(0, dst_pt)],
                        s_out.at[pl.ds(o0, dst_pt)])

    return agg


def _post1(s1, c1, b1):
    n = s1.shape[0] // 2
    def body(s_ref, c_ref, b_ref, o_ref):
        sa = s_ref[:n] + s_ref[n:]
        ca = c_ref[:n] + c_ref[n:]
        m = sa / jnp.maximum(ca, 1.0) + b_ref[...]
        o_ref[...] = jnp.maximum(m, 0.0)
    return pl.pallas_call(
        body, out_shape=jax.ShapeDtypeStruct((n, 8 * _L), jnp.float32),
    )(s1, c1, jnp.tile(b1, 8).reshape(1, 8 * _L))


def _final(s2, c2, w2, b2):
    n = s2.shape[0] // 2
    co = w2.shape[1]
    def body(s_ref, c_ref, w_ref, b_ref, o_ref):
        sa = s_ref[:n] + s_ref[n:]
        ca = c_ref[:n] + c_ref[n:]
        m = sa / jnp.maximum(ca, 1.0)
        h = jnp.dot(m, w_ref[...],
                    preferred_element_type=jnp.float32) + b_ref[...]
        mx = jnp.max(h, axis=1, keepdims=True)
        lse = jnp.log(jnp.sum(jnp.exp(h - mx), axis=1, keepdims=True))
        o_ref[...] = h - mx - lse
    return pl.pallas_call(
        body, out_shape=jax.ShapeDtypeStruct((n, co), jnp.float32),
    )(s2, c2, w2, b2.reshape(1, co))


def kernel(x, n_id, ei1_src, ei1_dst, ei2_src, ei2_dst, W1, b1, W2, b2):
    e1 = ei1_src.shape[0]
    e2 = ei2_src.shape[0]
    n1_dst, n2_dst = 16384, 4096

    n_nodes = x.shape[0]
    ht = _matmul_ht(x, W1).reshape(n_nodes, _L)

    src1 = ei1_src.astype(jnp.int32).reshape(e1 // 128, 128)
    dst1 = ei1_dst.astype(jnp.int32).reshape(e1 // 128, 128)
    src2 = ei2_src.astype(jnp.int32).reshape(e2 // 128, 128)
    dst2 = ei2_dst.astype(jnp.int32).reshape(e2 // 128, 128)
    nid = n_id.astype(jnp.int32)
    zeros_h = jnp.zeros((n1_dst, _L), jnp.float32)
    ones_h = jnp.ones((128, _L), jnp.float32)

    prep = _make_prep(nid.shape[0], e1 // 128, e2 // 128, n1_dst, n2_dst)
    idx1, c1, c2 = prep(src1, nid, dst1, dst2, zeros_h, ones_h)

    agg1 = _make_agg1(n1_dst, rows_pt=(e1 // 128) // _NW, kb=8)
    s1 = agg1(ht, idx1, dst1, zeros_h)

    h1p = _post1(s1.reshape(_NC * n1_dst // 8, 128),
                 c1.reshape(_NC * n1_dst // 8, 128), b1)
    h1 = h1p.reshape(n1_dst, _L)

    agg2 = _make_agg2(n2_dst, rows_pt=(e2 // 128) // _NW)
    s2 = agg2(h1, src2, dst2, zeros_h)

    return _final(s2, c2, W2, b2)


# speedup vs baseline: 1.0443x; 1.0443x over previous
import functools

import jax
import jax.numpy as jnp
from jax import lax
from jax.experimental import pallas as pl
from jax.experimental.pallas import tpu as pltpu
from jax.experimental.pallas import tpu_sc as plsc

_NC, _NS = 2, 16
_NW = _NC * _NS
_L = 16

_SC_PARAMS = pltpu.CompilerParams(
    needs_layout_passes=False, use_tc_tiling_on_sc=False)


def _matmul_ht(x, w):
    n, d = x.shape
    h = w.shape[1]
    bm = 4096
    def body(x_ref, w_ref, o_ref):
        r = jnp.dot(x_ref[...], w_ref[...],
                    preferred_element_type=jnp.float32)
        r3 = r.reshape(bm // 8, 8, h)
        o_ref[...] = jnp.concatenate([r3[:, k, :] for k in range(8)],
                                     axis=1)
    return pl.pallas_call(
        body,
        grid=((n + bm - 1) // bm,),
        in_specs=[pl.BlockSpec((bm, d), lambda i: (i, 0)),
                  pl.BlockSpec((d, h), lambda i: (0, 0))],
        out_specs=pl.BlockSpec((bm // 8, 8 * h), lambda i: (i, 0)),
        out_shape=jax.ShapeDtypeStruct((n // 8, 8 * h), jnp.float32),
    )(x, w)


def _make_prep(n_src, n_rows1, n_rows2, n1_dst, n2_dst):
    rows1_pt = n_rows1 // _NW
    rows2_pt = n_rows2 // _NW
    d1_pt = n1_dst // _NS
    d2_pt = n2_dst // _NS
    mesh = plsc.VectorSubcoreMesh(
        core_axis_name="c", subcore_axis_name="s",
        num_cores=_NC, num_subcores=_NS)
    scratch = [
        pltpu.VMEM((rows1_pt, 128), jnp.int32),
        pltpu.VMEM((rows1_pt, 128), jnp.int32),
        pltpu.VMEM((rows2_pt, 128), jnp.int32),
        pltpu.VMEM((128, _L), jnp.float32),
        pltpu.VMEM((d1_pt, _L), jnp.float32),
        pltpu.VMEM((n_src,), jnp.int32),
        pltpu.VMEM_SHARED((n1_dst, _L), jnp.float32),
        pltpu.VMEM_SHARED((n2_dst, _L), jnp.float32),
        pltpu.SemaphoreType.DMA,
    ]
    out_type = (jax.ShapeDtypeStruct((n_rows1, 128), jnp.int32),
                jax.ShapeDtypeStruct((_NC * n1_dst, _L), jnp.float32),
                jax.ShapeDtypeStruct((_NC * n2_dst, _L), jnp.float32))

    @functools.partial(pl.kernel, mesh=mesh, out_type=out_type,
                       scratch_types=scratch, compiler_params=_SC_PARAMS)
    def prep(srcr, nidr, dst1r, dst2r, zeros_h, ones_h,
             idx_out, c1_out, c2_out,
             src_v, dst1_v, dst2_v, ones_v, stage_v, nid_v,
             cnt1, cnt2, sem):
        cx = lax.axis_index("c")
        sx = lax.axis_index("s")
        w = cx * _NS + sx
        base1 = w * rows1_pt
        base2 = w * rows2_pt
        pltpu.sync_copy(ones_h, ones_v)
        pltpu.sync_copy(srcr.at[pl.ds(base1, rows1_pt)], src_v)
        pltpu.sync_copy(dst1r.at[pl.ds(base1, rows1_pt)], dst1_v)
        pltpu.sync_copy(dst2r.at[pl.ds(base2, rows2_pt)], dst2_v)
        pltpu.sync_copy(nidr, nid_v)
        z1 = sx * d1_pt
        z2 = sx * d2_pt
        pltpu.sync_copy(zeros_h.at[pl.ds(z1, d1_pt)],
                        cnt1.at[pl.ds(z1, d1_pt)])
        pltpu.sync_copy(zeros_h.at[pl.ds(z2, d2_pt)],
                        cnt2.at[pl.ds(z2, d2_pt)])
        plsc.subcore_barrier()

        for r in range(rows1_pt):
            pltpu.async_copy(ones_v, cnt1.at[dst1_v.at[r]], sem, add=True)
        for r in range(rows2_pt):
            pltpu.async_copy(ones_v, cnt2.at[dst2_v.at[r]], sem, add=True)

        def comp(r, carry):
            for t in range(8):
                vec = src_v[r, pl.ds(t * 16, 16)]
                src_v[r, pl.ds(t * 16, 16)] = plsc.load_gather(nid_v, [vec])
            return carry
        lax.fori_loop(0, rows1_pt, comp, 0)
        pltpu.sync_copy(src_v, idx_out.at[pl.ds(base1, rows1_pt)])

        for r in range(rows1_pt):
            pltpu.make_async_copy(ones_v, cnt1.at[dst1_v.at[r]],
                                  sem).wait()
        for r in range(rows2_pt):
            pltpu.make_async_copy(ones_v, cnt2.at[dst2_v.at[r]],
                                  sem).wait()
        plsc.subcore_barrier()

        o1 = cx * n1_dst + z1
        o2 = cx * n2_dst + z2
        pltpu.sync_copy(cnt1.at[pl.ds(z1, d1_pt)],
                        stage_v.at[pl.ds(0, d1_pt)])
        pltpu.sync_copy(stage_v.at[pl.ds(0, d1_pt)],
                        c1_out.at[pl.ds(o1, d1_pt)])
        pltpu.sync_copy(cnt2.at[pl.ds(z2, d2_pt)],
                        stage_v.at[pl.ds(0, d2_pt)])
        pltpu.sync_copy(stage_v.at[pl.ds(0, d2_pt)],
                        c2_out.at[pl.ds(o2, d2_pt)])

    return prep


def _make_agg1(n_dst, rows_pt, kb):
    n_chunks = rows_pt // kb
    dst_pt = n_dst // _NS
    mesh = plsc.VectorSubcoreMesh(
        core_axis_name="c", subcore_axis_name="s",
        num_cores=_NC, num_subcores=_NS)

    scratch = [
        pltpu.VMEM((rows_pt, 128), jnp.int32),
        pltpu.VMEM((rows_pt, 128), jnp.int32),
        pltpu.VMEM((kb * 128, _L), jnp.float32),
        pltpu.VMEM((kb * 128, _L), jnp.float32),
        pltpu.VMEM((kb * 128, _L), jnp.float32),
        pltpu.VMEM((kb * 128, _L), jnp.float32),
        pltpu.VMEM_SHARED((n_dst, _L), jnp.float32),
        pltpu.SemaphoreType.DMA,
        pltpu.SemaphoreType.DMA,
    ]
    out_type = jax.ShapeDtypeStruct((_NC * n_dst, _L), jnp.float32)

    @functools.partial(pl.kernel, mesh=mesh, out_type=out_type,
                       scratch_types=scratch, compiler_params=_SC_PARAMS)
    def agg(table, idxr, dstr, zeros_h, s_out,
            src_v, dst_v, rows_0, rows_1, rows_2, rows_3, acc,
            gsem, ssem):
        cx = lax.axis_index("c")
        sx = lax.axis_index("s")
        w = cx * _NS + sx
        base = w * rows_pt

        pltpu.sync_copy(idxr.at[pl.ds(base, rows_pt)], src_v)
        pltpu.sync_copy(dstr.at[pl.ds(base, rows_pt)], dst_v)
        z0 = sx * dst_pt
        pltpu.sync_copy(zeros_h.at[pl.ds(z0, dst_pt)],
                        acc.at[pl.ds(z0, dst_pt)])
        plsc.subcore_barrier()

        def fire_g(c, buf):
            for r in range(kb):
                pltpu.async_copy(table.at[src_v.at[c * kb + r]],
                                 buf.at[pl.ds(r * 128, 128)], gsem)

        def drain_g(c, buf):
            for r in range(kb):
                pltpu.make_async_copy(
                    table.at[src_v.at[c * kb + r]],
                    buf.at[pl.ds(r * 128, 128)], gsem).wait()

        def fire_s(c, buf):
            for r in range(kb):
                pltpu.async_copy(buf.at[pl.ds(r * 128, 128)],
                                 acc.at[dst_v.at[c * kb + r]], ssem,
                                 add=True)

        def drain_s(c, buf):
            for r in range(kb):
                pltpu.make_async_copy(
                    buf.at[pl.ds(r * 128, 128)],
                    acc.at[dst_v.at[c * kb + r]], ssem).wait()

        bufs = (rows_0, rows_1, rows_2, rows_3)
        fire_g(0, bufs[0])
        fire_g(1, bufs[1])
        fire_g(2, bufs[2])
        drain_g(0, bufs[0])
        fire_s(0, bufs[0])
        fire_g(3, bufs[3])
        drain_g(1, bufs[1])
        fire_s(1, bufs[1])

        def quad(i, carry):
            for j in range(4):
                c = 2 + 4 * i + j
                b = bufs[(2 + j) % 4]
                drain_s(c - 2, bufs[j % 4])
                fire_g(c + 2, bufs[j % 4])
                drain_g(c, b)
                fire_s(c, b)
            return carry
        lax.fori_loop(0, (n_chunks - 4) // 4, quad, 0)

        c0 = n_chunks - 2
        drain_s(c0 - 2, bufs[0])
        drain_g(c0, bufs[2])
        fire_s(c0, bufs[2])
        drain_s(c0 - 1, bufs[1])
        drain_g(c0 + 1, bufs[3])
        fire_s(c0 + 1, bufs[3])
        drain_s(c0, bufs[2])
        drain_s(c0 + 1, bufs[3])
        plsc.subcore_barrier()

        o0 = cx * n_dst + sx * dst_pt
        pltpu.sync_copy(acc.at[pl.ds(z0, dst_pt)],
                        rows_0.at[pl.ds(0, dst_pt)])
        pltpu.sync_copy(rows_0.at[pl.ds(0, dst_pt)],
                        s_out.at[pl.ds(o0, dst_pt)])

    return agg


def _make_agg2(n_dst, rows_pt):
    dst_pt = n_dst // _NS
    rows_cap = max(rows_pt * 128, dst_pt)
    mesh = plsc.VectorSubcoreMesh(
        core_axis_name="c", subcore_axis_name="s",
        num_cores=_NC, num_subcores=_NS)

    scratch = [
        pltpu.VMEM((rows_pt, 128), jnp.int32),
        pltpu.VMEM((rows_pt, 128), jnp.int32),
        pltpu.VMEM((rows_cap, _L), jnp.float32),
        pltpu.VMEM_SHARED((n_dst, _L), jnp.float32),
        pltpu.SemaphoreType.DMA,
        pltpu.SemaphoreType.DMA,
    ]
    out_type = jax.ShapeDtypeStruct((_NC * n_dst, _L), jnp.float32)

    @functools.partial(pl.kernel, mesh=mesh, out_type=out_type,
                       scratch_types=scratch, compiler_params=_SC_PARAMS)
    def agg(table, srcr, dstr, zeros_h, s_out,
            src_v, dst_v, rows_v, acc, gsem, ssem):
        cx = lax.axis_index("c")
        sx = lax.axis_index("s")
        w = cx * _NS + sx
        base = w * rows_pt

        pltpu.sync_copy(srcr.at[pl.ds(base, rows_pt)], src_v)
        pltpu.sync_copy(dstr.at[pl.ds(base, rows_pt)], dst_v)
        z0 = sx * dst_pt
        pltpu.sync_copy(zeros_h.at[pl.ds(z0, dst_pt)],
                        acc.at[pl.ds(z0, dst_pt)])
        plsc.subcore_barrier()

        for r in range(rows_pt):
            pltpu.async_copy(table.at[src_v.at[r]],
                             rows_v.at[pl.ds(r * 128, 128)], gsem)
        for r in range(rows_pt):
            pltpu.make_async_copy(table.at[src_v.at[r]],
                                  rows_v.at[pl.ds(r * 128, 128)],
                                  gsem).wait()
        for r in range(rows_pt):
            pltpu.async_copy(rows_v.at[pl.ds(r * 128, 128)],
                             acc.at[dst_v.at[r]], ssem, add=True)
        for r in range(rows_pt):
            pltpu.make_async_copy(rows_v.at[pl.ds(r * 128, 128)],
                                  acc.at[dst_v.at[r]], ssem).wait()
        plsc.subcore_barrier()

        o0 = cx * n_dst + sx * dst_pt
        pltpu.sync_copy(acc.at[pl.ds(z0, dst_pt)],
                        rows_v.at[pl.ds(0, dst_pt)])
        pltpu.sync_copy(rows_v.at[pl.ds(0, dst_pt)],
                        s_out.at[pl.ds(o0, dst_pt)])

    return agg


def _post1(s1, c1, b1):
    n = s1.shape[0] // 2
    def body(s_ref, c_ref, b_ref, o_ref):
        sa = s_ref[:n] + s_ref[n:]
        ca = c_ref[:n] + c_ref[n:]
        m = sa / jnp.maximum(ca, 1.0) + b_ref[...]
        o_ref[...] = jnp.maximum(m, 0.0)
    return pl.pallas_call(
        body, out_shape=jax.ShapeDtypeStruct((n, 8 * _L), jnp.float32),
    )(s1, c1, jnp.tile(b1, 8).reshape(1, 8 * _L))


def _final(s2, c2, w2, b2):
    n = s2.shape[0] // 2
    co = w2.shape[1]
    def body(s_ref, c_ref, w_ref, b_ref, o_ref):
        sa = s_ref[:n] + s_ref[n:]
        ca = c_ref[:n] + c_ref[n:]
        m = sa / jnp.maximum(ca, 1.0)
        h = jnp.dot(m, w_ref[...],
                    preferred_element_type=jnp.float32) + b_ref[...]
        mx = jnp.max(h, axis=1, keepdims=True)
        lse = jnp.log(jnp.sum(jnp.exp(h - mx), axis=1, keepdims=True))
        o_ref[...] = h - mx - lse
    return pl.pallas_call(
        body, out_shape=jax.ShapeDtypeStruct((n, co), jnp.float32),
    )(s2, c2, w2, b2.reshape(1, co))


def kernel(x, n_id, ei1_src, ei1_dst, ei2_src, ei2_dst, W1, b1, W2, b2):
    e1 = ei1_src.shape[0]
    e2 = ei2_src.shape[0]
    n1_dst, n2_dst = 16384, 4096

    n_nodes = x.shape[0]
    ht = _matmul_ht(x, W1).reshape(n_nodes, _L)

    src1 = ei1_src.astype(jnp.int32).reshape(e1 // 128, 128)
    dst1 = ei1_dst.astype(jnp.int32).reshape(e1 // 128, 128)
    src2 = ei2_src.astype(jnp.int32).reshape(e2 // 128, 128)
    dst2 = ei2_dst.astype(jnp.int32).reshape(e2 // 128, 128)
    nid = n_id.astype(jnp.int32)
    zeros_h = jnp.zeros((n1_dst, _L), jnp.float32)
    ones_h = jnp.ones((128, _L), jnp.float32)

    prep = _make_prep(nid.shape[0], e1 // 128, e2 // 128, n1_dst, n2_dst)
    idx1, c1, c2 = prep(src1, nid, dst1, dst2, zeros_h, ones_h)

    agg1 = _make_agg1(n1_dst, rows_pt=(e1 // 128) // _NW, kb=8)
    s1 = agg1(ht, idx1, dst1, zeros_h)

    h1p = _post1(s1.reshape(_NC * n1_dst // 8, 128),
                 c1.reshape(_NC * n1_dst // 8, 128), b1)
    h1 = h1p.reshape(n1_dst, _L)

    agg2 = _make_agg2(n2_dst, rows_pt=(e2 // 128) // _NW)
    s2 = agg2(h1, src2, dst2, zeros_h)

    return _final(s2, c2, W2, b2)


# speedup vs baseline: 1.0868x; 1.0407x over previous
import functools

import jax
import jax.numpy as jnp
from jax import lax
from jax.experimental import pallas as pl
from jax.experimental.pallas import tpu as pltpu
from jax.experimental.pallas import tpu_sc as plsc

_NC, _NS = 2, 16
_NW = _NC * _NS
_L = 16

_SC_PARAMS = pltpu.CompilerParams(
    needs_layout_passes=False, use_tc_tiling_on_sc=False)


def _matmul_ht(x, w):
    n, d = x.shape
    h = w.shape[1]
    bm = 8192
    def body(x_ref, w_ref, o_ref):
        r = jnp.dot(x_ref[...], w_ref[...],
                    preferred_element_type=jnp.float32)
        r3 = r.reshape(bm // 8, 8, h)
        o_ref[...] = jnp.concatenate([r3[:, k, :] for k in range(8)],
                                     axis=1)
    return pl.pallas_call(
        body,
        grid=((n + bm - 1) // bm,),
        in_specs=[pl.BlockSpec((bm, d), lambda i: (i, 0)),
                  pl.BlockSpec((d, h), lambda i: (0, 0))],
        out_specs=pl.BlockSpec((bm // 8, 8 * h), lambda i: (i, 0)),
        out_shape=jax.ShapeDtypeStruct((n // 8, 8 * h), jnp.float32),
    )(x, w)


def _make_prep(n_src, n_rows1, n_rows2, n1_dst, n2_dst):
    rows1_pt = n_rows1 // _NW
    rows2_pt = n_rows2 // _NW
    d1_pt = n1_dst // _NS
    d2_pt = n2_dst // _NS
    mesh = plsc.VectorSubcoreMesh(
        core_axis_name="c", subcore_axis_name="s",
        num_cores=_NC, num_subcores=_NS)
    scratch = [
        pltpu.VMEM((rows1_pt, 128), jnp.int32),
        pltpu.VMEM((rows1_pt, 128), jnp.int32),
        pltpu.VMEM((rows2_pt, 128), jnp.int32),
        pltpu.VMEM((128, _L), jnp.float32),
        pltpu.VMEM((d1_pt, _L), jnp.float32),
        pltpu.VMEM((n_src,), jnp.int32),
        pltpu.VMEM_SHARED((n1_dst, _L), jnp.float32),
        pltpu.VMEM_SHARED((n2_dst, _L), jnp.float32),
        pltpu.SemaphoreType.DMA,
    ]
    out_type = (jax.ShapeDtypeStruct((n_rows1, 128), jnp.int32),
                jax.ShapeDtypeStruct((_NC * n1_dst, _L), jnp.float32),
                jax.ShapeDtypeStruct((_NC * n2_dst, _L), jnp.float32))

    @functools.partial(pl.kernel, mesh=mesh, out_type=out_type,
                       scratch_types=scratch, compiler_params=_SC_PARAMS)
    def prep(srcr, nidr, dst1r, dst2r, zeros_h, ones_h,
             idx_out, c1_out, c2_out,
             src_v, dst1_v, dst2_v, ones_v, stage_v, nid_v,
             cnt1, cnt2, sem):
        cx = lax.axis_index("c")
        sx = lax.axis_index("s")
        w = cx * _NS + sx
        base1 = w * rows1_pt
        base2 = w * rows2_pt
        pltpu.sync_copy(ones_h, ones_v)
        pltpu.sync_copy(srcr.at[pl.ds(base1, rows1_pt)], src_v)
        pltpu.sync_copy(dst1r.at[pl.ds(base1, rows1_pt)], dst1_v)
        pltpu.sync_copy(dst2r.at[pl.ds(base2, rows2_pt)], dst2_v)
        pltpu.sync_copy(nidr, nid_v)
        z1 = sx * d1_pt
        z2 = sx * d2_pt
        pltpu.sync_copy(zeros_h.at[pl.ds(z1, d1_pt)],
                        cnt1.at[pl.ds(z1, d1_pt)])
        pltpu.sync_copy(zeros_h.at[pl.ds(z2, d2_pt)],
                        cnt2.at[pl.ds(z2, d2_pt)])
        plsc.subcore_barrier()

        for r in range(rows1_pt):
            pltpu.async_copy(ones_v, cnt1.at[dst1_v.at[r]], sem, add=True)
        for r in range(rows2_pt):
            pltpu.async_copy(ones_v, cnt2.at[dst2_v.at[r]], sem, add=True)

        def comp(r, carry):
            for t in range(8):
                vec = src_v[r, pl.ds(t * 16, 16)]
                src_v[r, pl.ds(t * 16, 16)] = plsc.load_gather(nid_v, [vec])
            return carry
        lax.fori_loop(0, rows1_pt, comp, 0)
        pltpu.sync_copy(src_v, idx_out.at[pl.ds(base1, rows1_pt)])

        for r in range(rows1_pt):
            pltpu.make_async_copy(ones_v, cnt1.at[dst1_v.at[r]],
                                  sem).wait()
        for r in range(rows2_pt):
            pltpu.make_async_copy(ones_v, cnt2.at[dst2_v.at[r]],
                                  sem).wait()
        plsc.subcore_barrier()

        o1 = cx * n1_dst + z1
        o2 = cx * n2_dst + z2
        pltpu.sync_copy(cnt1.at[pl.ds(z1, d1_pt)],
                        stage_v.at[pl.ds(0, d1_pt)])
        pltpu.sync_copy(stage_v.at[pl.ds(0, d1_pt)],
                        c1_out.at[pl.ds(o1, d1_pt)])
        pltpu.sync_copy(cnt2.at[pl.ds(z2, d2_pt)],
                        stage_v.at[pl.ds(0, d2_pt)])
        pltpu.sync_copy(stage_v.at[pl.ds(0, d2_pt)],
                        c2_out.at[pl.ds(o2, d2_pt)])

    return prep


def _make_agg1(n_dst, rows_pt, kb):
    n_chunks = rows_pt // kb
    dst_pt = n_dst // _NS
    mesh = plsc.VectorSubcoreMesh(
        core_axis_name="c", subcore_axis_name="s",
        num_cores=_NC, num_subcores=_NS)

    scratch = [
        pltpu.VMEM((rows_pt, 128), jnp.int32),
        pltpu.VMEM((rows_pt, 128), jnp.int32),
        pltpu.VMEM((kb * 128, _L), jnp.float32),
        pltpu.VMEM((kb * 128, _L), jnp.float32),
        pltpu.VMEM((kb * 128, _L), jnp.float32),
        pltpu.VMEM((kb * 128, _L), jnp.float32),
        pltpu.VMEM_SHARED((n_dst, _L), jnp.float32),
        pltpu.SemaphoreType.DMA,
        pltpu.SemaphoreType.DMA,
    ]
    out_type = jax.ShapeDtypeStruct((_NC * n_dst, _L), jnp.float32)

    @functools.partial(pl.kernel, mesh=mesh, out_type=out_type,
                       scratch_types=scratch, compiler_params=_SC_PARAMS)
    def agg(table, idxr, dstr, zeros_h, s_out,
            src_v, dst_v, rows_0, rows_1, rows_2, rows_3, acc,
            gsem, ssem):
        cx = lax.axis_index("c")
        sx = lax.axis_index("s")
        w = cx * _NS + sx
        base = w * rows_pt

        pltpu.sync_copy(idxr.at[pl.ds(base, rows_pt)], src_v)
        pltpu.sync_copy(dstr.at[pl.ds(base, rows_pt)], dst_v)
        z0 = sx * dst_pt
        pltpu.sync_copy(zeros_h.at[pl.ds(z0, dst_pt)],
                        acc.at[pl.ds(z0, dst_pt)])
        plsc.subcore_barrier()

        def fire_g(c, buf):
            for r in range(kb):
                pltpu.async_copy(table.at[src_v.at[c * kb + r]],
                                 buf.at[pl.ds(r * 128, 128)], gsem)

        def drain_g(c, buf):
            for r in range(kb):
                pltpu.make_async_copy(
                    table.at[src_v.at[c * kb + r]],
                    buf.at[pl.ds(r * 128, 128)], gsem).wait()

        def fire_s(c, buf):
            for r in range(kb):
                pltpu.async_copy(buf.at[pl.ds(r * 128, 128)],
                                 acc.at[dst_v.at[c * kb + r]], ssem,
                                 add=True)

        def drain_s(c, buf):
            for r in range(kb):
                pltpu.make_async_copy(
                    buf.at[pl.ds(r * 128, 128)],
                    acc.at[dst_v.at[c * kb + r]], ssem).wait()

        bufs = (rows_0, rows_1, rows_2, rows_3)
        fire_g(0, bufs[0])
        fire_g(1, bufs[1])
        fire_g(2, bufs[2])
        drain_g(0, bufs[0])
        fire_s(0, bufs[0])
        fire_g(3, bufs[3])
        drain_g(1, bufs[1])
        fire_s(1, bufs[1])

        def quad(i, carry):
            for j in range(4):
                c = 2 + 4 * i + j
                b = bufs[(2 + j) % 4]
                drain_s(c - 2, bufs[j % 4])
                fire_g(c + 2, bufs[j % 4])
                drain_g(c, b)
                fire_s(c, b)
            return carry
        lax.fori_loop(0, (n_chunks - 4) // 4, quad, 0)

        c0 = n_chunks - 2
        drain_s(c0 - 2, bufs[0])
        drain_g(c0, bufs[2])
        fire_s(c0, bufs[2])
        drain_s(c0 - 1, bufs[1])
        drain_g(c0 + 1, bufs[3])
        fire_s(c0 + 1, bufs[3])
        drain_s(c0, bufs[2])
        drain_s(c0 + 1, bufs[3])
        plsc.subcore_barrier()

        o0 = cx * n_dst + sx * dst_pt
        pltpu.sync_copy(acc.at[pl.ds(z0, dst_pt)],
                        rows_0.at[pl.ds(0, dst_pt)])
        pltpu.sync_copy(rows_0.at[pl.ds(0, dst_pt)],
                        s_out.at[pl.ds(o0, dst_pt)])

    return agg


def _make_agg2(n_dst, rows_pt):
    dst_pt = n_dst // _NS
    rows_cap = max(rows_pt * 128, dst_pt)
    mesh = plsc.VectorSubcoreMesh(
        core_axis_name="c", subcore_axis_name="s",
        num_cores=_NC, num_subcores=_NS)

    scratch = [
        pltpu.VMEM((rows_pt, 128), jnp.int32),
        pltpu.VMEM((rows_pt, 128), jnp.int32),
        pltpu.VMEM((rows_cap, _L), jnp.float32),
        pltpu.VMEM_SHARED((n_dst, _L), jnp.float32),
        pltpu.SemaphoreType.DMA,
        pltpu.SemaphoreType.DMA,
    ]
    out_type = jax.ShapeDtypeStruct((_NC * n_dst, _L), jnp.float32)

    @functools.partial(pl.kernel, mesh=mesh, out_type=out_type,
                       scratch_types=scratch, compiler_params=_SC_PARAMS)
    def agg(table, srcr, dstr, zeros_h, s_out,
            src_v, dst_v, rows_v, acc, gsem, ssem):
        cx = lax.axis_index("c")
        sx = lax.axis_index("s")
        w = cx * _NS + sx
        base = w * rows_pt

        pltpu.sync_copy(srcr.at[pl.ds(base, rows_pt)], src_v)
        pltpu.sync_copy(dstr.at[pl.ds(base, rows_pt)], dst_v)
        z0 = sx * dst_pt
        pltpu.sync_copy(zeros_h.at[pl.ds(z0, dst_pt)],
                        acc.at[pl.ds(z0, dst_pt)])
        plsc.subcore_barrier()

        for r in range(rows_pt):
            pltpu.async_copy(table.at[src_v.at[r]],
                             rows_v.at[pl.ds(r * 128, 128)], gsem)
        for r in range(rows_pt):
            pltpu.make_async_copy(table.at[src_v.at[r]],
                                  rows_v.at[pl.ds(r * 128, 128)],
                                  gsem).wait()
        for r in range(rows_pt):
            pltpu.async_copy(rows_v.at[pl.ds(r * 128, 128)],
                             acc.at[dst_v.at[r]], ssem, add=True)
        for r in range(rows_pt):
            pltpu.make_async_copy(rows_v.at[pl.ds(r * 128, 128)],
                                  acc.at[dst_v.at[r]], ssem).wait()
        plsc.subcore_barrier()

        o0 = cx * n_dst + sx * dst_pt
        pltpu.sync_copy(acc.at[pl.ds(z0, dst_pt)],
                        rows_v.at[pl.ds(0, dst_pt)])
        pltpu.sync_copy(rows_v.at[pl.ds(0, dst_pt)],
                        s_out.at[pl.ds(o0, dst_pt)])

    return agg


def _post1(s1, c1, b1):
    n = s1.shape[0] // 2
    def body(s_ref, c_ref, b_ref, o_ref):
        sa = s_ref[:n] + s_ref[n:]
        ca = c_ref[:n] + c_ref[n:]
        m = sa / jnp.maximum(ca, 1.0) + b_ref[...]
        o_ref[...] = jnp.maximum(m, 0.0)
    return pl.pallas_call(
        body, out_shape=jax.ShapeDtypeStruct((n, 8 * _L), jnp.float32),
    )(s1, c1, jnp.tile(b1, 8).reshape(1, 8 * _L))


def _final(s2, c2, w2, b2):
    n = s2.shape[0] // 2
    co = w2.shape[1]
    def body(s_ref, c_ref, w_ref, b_ref, o_ref):
        sa = s_ref[:n] + s_ref[n:]
        ca = c_ref[:n] + c_ref[n:]
        m = sa / jnp.maximum(ca, 1.0)
        h = jnp.dot(m, w_ref[...],
                    preferred_element_type=jnp.float32) + b_ref[...]
        mx = jnp.max(h, axis=1, keepdims=True)
        lse = jnp.log(jnp.sum(jnp.exp(h - mx), axis=1, keepdims=True))
        o_ref[...] = h - mx - lse
    return pl.pallas_call(
        body, out_shape=jax.ShapeDtypeStruct((n, co), jnp.float32),
    )(s2, c2, w2, b2.reshape(1, co))


def kernel(x, n_id, ei1_src, ei1_dst, ei2_src, ei2_dst, W1, b1, W2, b2):
    e1 = ei1_src.shape[0]
    e2 = ei2_src.shape[0]
    n1_dst, n2_dst = 16384, 4096

    n_nodes = x.shape[0]
    ht = _matmul_ht(x, W1).reshape(n_nodes, _L)

    src1 = ei1_src.astype(jnp.int32).reshape(e1 // 128, 128)
    dst1 = ei1_dst.astype(jnp.int32).reshape(e1 // 128, 128)
    src2 = ei2_src.astype(jnp.int32).reshape(e2 // 128, 128)
    dst2 = ei2_dst.astype(jnp.int32).reshape(e2 // 128, 128)
    nid = n_id.astype(jnp.int32)
    zeros_h = jnp.zeros((n1_dst, _L), jnp.float32)
    ones_h = jnp.ones((128, _L), jnp.float32)

    prep = _make_prep(nid.shape[0], e1 // 128, e2 // 128, n1_dst, n2_dst)
    idx1, c1, c2 = prep(src1, nid, dst1, dst2, zeros_h, ones_h)

    agg1 = _make_agg1(n1_dst, rows_pt=(e1 // 128) // _NW, kb=8)
    s1 = agg1(ht, idx1, dst1, zeros_h)

    h1p = _post1(s1.reshape(_NC * n1_dst // 8, 128),
                 c1.reshape(_NC * n1_dst // 8, 128), b1)
    h1 = h1p.reshape(n1_dst, _L)

    agg2 = _make_agg2(n2_dst, rows_pt=(e2 // 128) // _NW)
    s2 = agg2(h1, src2, dst2, zeros_h)

    return _final(s2, c2, W2, b2)
